# Initial kernel scaffold; baseline (speedup 1.0000x reference)
import jax
import jax.numpy as jnp
from jax.experimental import pallas as pl


def kernel(x, var_embed, W_t, b_t, W_q, b_q, W_k, b_k, W_g, b_g):
    raise NotImplementedError("write your pallas kernel here")


# speedup vs baseline: 5.4213x; 5.4213x over previous
import functools
import math

import numpy as np
import jax
import jax.numpy as jnp
from jax import lax
from jax.experimental import pallas as pl
from jax.experimental.pallas import tpu as pltpu
from jax.experimental.pallas import tpu_sc as plsc

B, L, N, H = 8, 32, 2048, 16
TOPK_DYN = min(N, max(16, int(math.log2(N) * 3)))
TOPK_STA = max(32, int(math.sqrt(N) * 2))
KC = 96
KCP = 112
U = B * N
NC2 = N // 8
I32 = jnp.int32
NEG32 = -(2 ** 31)
POS32 = 2 ** 31 - 1

_NN1 = np.float32(N * N) - np.float32(1.0)
_QLO = np.float32(np.float32(0.05) * _NN1)
_QHI = np.float32(np.float32(1.0 - 0.05) * _NN1)
K_LO = int(np.floor(_QLO))
K_HI = int(np.floor(_QHI))
HW_LO = np.float32(_QLO - np.float32(K_LO))
LW_LO = np.float32(np.float32(1.0) - HW_LO)
HW_HI = np.float32(_QHI - np.float32(K_HI))
LW_HI = np.float32(np.float32(1.0) - HW_HI)

NW = 32
RPW = N // NW

_pm = np.zeros((N, NC2), np.float32)
_pm[np.arange(N), np.arange(N) // 8] = 2.0 ** (np.arange(N) % 8)
PACKM = _pm

_pt = np.zeros((256, 16), np.int32)
for _code in range(256):
    _bits = [i for i in range(8) if (_code >> i) & 1]
    _pt[_code, :len(_bits)] = _bits
    _pt[_code, 8] = len(_bits)
PTAB = _pt

_GDN = lax.GatherDimensionNumbers(offset_dims=(), collapsed_slice_dims=(0,),
                                  start_index_map=(0,))


def _gather16(vec, idx):
    return lax.gather(vec, idx[:, None], _GDN, (1,),
                      mode=lax.GatherScatterMode.PROMISE_IN_BOUNDS)


def _mid_floor(lo, hi):
    return (lo >> 1) + (hi >> 1) + (lo & hi & 1)


def _key_to_f32(k):
    ib = jnp.where(k >= 0, k, k ^ jnp.int32(0x7FFFFFFF))
    return lax.bitcast_convert_type(ib, jnp.float32)


def _tc_sim(xv_ref, ve_ref, wt0_ref, wt1_ref, bt_ref, wqt_ref, bq_ref,
            wkt_ref, bk_ref, wg_ref, bg_ref,
            s_ref, q_ref, k_ref, imp_ref):
    xv = xv_ref[...]
    xsum = jnp.zeros((N, L), jnp.float32)
    for b in range(B):
        xsum = xsum + xv[b]
    x_avg = xsum / np.float32(B)
    ssq = jnp.zeros((N, L), jnp.float32)
    for b in range(B):
        d = xv[b] - x_avg
        ssq = ssq + d * d
    x_std = jnp.sqrt(ssq / np.float32(B - 1)) + np.float32(1e-05)
    normed = x_avg / x_std
    nb = normed.astype(jnp.bfloat16)
    s_ref[...] = lax.dot_general(nb, nb, (((1,), (1,)), ((), ())),
                                 preferred_element_type=jnp.float32)

    ve = ve_ref[...]
    wt0 = wt0_ref[...]
    wt1 = wt1_ref[...]
    bt = bt_ref[...]
    wg = wg_ref[...]
    wg16 = wg.astype(jnp.bfloat16).astype(jnp.float32)
    wt0_16 = wt0.astype(jnp.bfloat16).astype(jnp.float32)
    wt1_16 = wt1.astype(jnp.bfloat16).astype(jnp.float32)
    bg = bg_ref[0, 0]
    for b in range(B):
        xb = xv[b]
        mean_b = jnp.mean(xb, axis=1, keepdims=True)
        d = xb - mean_b
        var_b = jnp.sum(d * d, axis=1, keepdims=True) / np.float32(L - 1)
        std_b = jnp.sqrt(var_b + np.float32(1e-05))
        mb16 = mean_b.astype(jnp.bfloat16).astype(jnp.float32)
        sb16 = std_b.astype(jnp.bfloat16).astype(jnp.float32)
        te = mb16 * wt0_16 + sb16 * wt1_16 + bt
        vf = jnp.concatenate([ve, te], axis=1)
        vfb = vf.astype(jnp.bfloat16)
        qb = jnp.dot(vfb, wqt_ref[...].astype(jnp.bfloat16),
                     preferred_element_type=jnp.float32) + bq_ref[...]
        kb = jnp.dot(vfb, wkt_ref[...].astype(jnp.bfloat16),
                     preferred_element_type=jnp.float32) + bk_ref[...]
        vf16 = vf.astype(jnp.bfloat16).astype(jnp.float32)
        impb = jax.nn.sigmoid(
            jnp.sum(vf16 * wg16, axis=1, keepdims=True) + bg)
        q_ref[b] = qb
        k_ref[b] = kb
        imp_ref[b] = impb


_QB = 256


def _tc_quant(sraw_ref, s_ref):
    def count2(f1, f2):
        def blk(j, acc):
            a1, a2 = acc
            sb = sraw_ref[pl.ds(j * _QB, _QB), :]
            a1 = a1 + jnp.sum((sb <= f1).astype(I32))
            a2 = a2 + jnp.sum((sb <= f2).astype(I32))
            return a1, a2
        return lax.fori_loop(0, N // _QB, blk, (jnp.int32(0), jnp.int32(0)))

    def qbody(i, c):
        lo1, hi1, lo2, hi2 = c
        mi1 = _mid_floor(lo1, hi1)
        mi2 = _mid_floor(lo2, hi2)
        c1, c2 = count2(_key_to_f32(mi1), _key_to_f32(mi2))
        ok1 = c1 >= (K_LO + 1)
        ok2 = c2 >= (K_HI + 1)
        return (jnp.where(ok1, lo1, mi1 + 1), jnp.where(ok1, mi1, hi1),
                jnp.where(ok2, lo2, mi2 + 1), jnp.where(ok2, mi2, hi2))

    t1, _, t2, _ = lax.fori_loop(
        0, 32, qbody,
        (jnp.int32(NEG32), jnp.int32(POS32), jnp.int32(NEG32),
         jnp.int32(POS32)))
    t1f = _key_to_f32(t1)
    t2f = _key_to_f32(t2)
    cle1, cle2 = count2(t1f, t2f)

    def sblk(j, acc):
        s1, s2 = acc
        sb = sraw_ref[pl.ds(j * _QB, _QB), :]
        s1 = jnp.minimum(s1, jnp.min(
            jnp.where(sb > t1f, sb, np.float32(np.inf))))
        s2 = jnp.minimum(s2, jnp.min(
            jnp.where(sb > t2f, sb, np.float32(np.inf))))
        return s1, s2

    succ1, succ2 = lax.fori_loop(
        0, N // _QB, sblk, (np.float32(np.inf), np.float32(np.inf)))
    v1b = jnp.where(cle1 >= K_LO + 2, t1f, succ1)
    v2b = jnp.where(cle2 >= K_HI + 2, t2f, succ2)
    low_q = t1f * LW_LO + v1b * HW_LO
    high_q = t2f * LW_HI + v2b * HW_HI

    def wblk(j, carry):
        sb = sraw_ref[pl.ds(j * _QB, _QB), :]
        sc = jnp.clip(sb, low_q, high_q)
        rr = lax.broadcasted_iota(I32, (_QB, N), 0) + j * _QB
        ccb = lax.broadcasted_iota(I32, (_QB, N), 1)
        sc = jnp.where(rr == ccb, np.float32(0.0), sc)
        s_ref[pl.ds(j * _QB, _QB), :] = jnp.maximum(sc, np.float32(0.0))
        return carry

    lax.fori_loop(0, N // _QB, wblk, 0)


_MB = 256


def _tc_mask(s_ref, pk_ref, codes_ref):
    s = s_ref[...]
    cc = lax.broadcasted_iota(I32, (_MB, N), 1)

    def body(i, c):
        lo, hi = c
        mid = _mid_floor(lo, hi) + ((lo ^ hi) & 1)
        midf = lax.bitcast_convert_type(mid, jnp.float32)
        cnt = jnp.sum((s >= midf).astype(I32), axis=1, keepdims=True)
        ok = cnt >= TOPK_STA
        return jnp.where(ok, mid, lo), jnp.where(ok, hi, mid - 1)

    lo = jnp.zeros((_MB, 1), I32)
    hi = jnp.full((_MB, 1), POS32, I32)
    lo, hi = lax.fori_loop(0, 32, body, (lo, hi))
    tf = lax.bitcast_convert_type(lo, jnp.float32)
    c_gt = jnp.sum((s > tf).astype(I32), axis=1, keepdims=True)
    r = TOPK_STA - c_gt

    def body2(i, c):
        lo2, hi2 = c
        mid = _mid_floor(lo2, hi2)
        d = jnp.sum(((s == tf) & (cc <= mid)).astype(I32), axis=1,
                    keepdims=True)
        ok = d >= r
        return jnp.where(ok, lo2, mid + 1), jnp.where(ok, mid, hi2)

    lo2 = jnp.zeros((_MB, 1), I32)
    hi2 = jnp.full((_MB, 1), N - 1, I32)
    lo2, hi2 = lax.fori_loop(0, 12, body2, (lo2, hi2))
    mask = (s > tf) | ((s == tf) & (cc <= lo2))
    codes = jnp.dot(mask.astype(jnp.float32), pk_ref[...],
                    preferred_element_type=jnp.float32)
    codes_ref[...] = codes.astype(I32)


def _sc_compact_body(codes_hbm, ptab_hbm, cand_hbm, cbuf, crows, ptabv):
    wid = lax.axis_index("s") * 2 + lax.axis_index("c")
    base = wid * RPW
    pltpu.sync_copy(ptab_hbm, ptabv)
    lane = lax.iota(I32, 16)
    zi = lane * 0
    for g in range(RPW // 16):
        row0 = base + g * 16
        pltpu.sync_copy(codes_hbm.at[pl.ds(row0, 16)], cbuf)

        def row_body(rl, carry):
            def octo(oc, run):
                cv = cbuf[rl, pl.ds(oc * 16, 16)]
                for c2 in range(8):
                    code_lo = cv[c2 * 2]
                    code_hi = cv[c2 * 2 + 1]
                    plo = ptabv[code_lo, pl.ds(0, 16)]
                    phi = ptabv[code_hi, pl.ds(0, 16)]
                    cnt_lo = plo[8]
                    clsp = zi + cnt_lo
                    mg = jnp.where(
                        lane < clsp, plo,
                        _gather16(phi, jnp.maximum(lane - clsp, 0)) + 8)
                    crows[rl, pl.ds(run, 16)] = mg + ((oc * 8 + c2) * 16)
                    run = run + cnt_lo + phi[8]
                return run

            lax.fori_loop(0, 16, octo, jnp.int32(0))
            crows[rl, pl.ds(90, 16)] = jnp.zeros((16,), I32)
            crows[rl, pl.ds(96, 16)] = jnp.zeros((16,), I32)
            return carry

        lax.fori_loop(0, 16, row_body, 0)
        pltpu.sync_copy(crows, cand_hbm.at[pl.ds(row0, 16)])


def _sc_gather_body(cand_hbm, kf_hbm, kg_hbm, candb, ridx, kgb, sem):
    wid = lax.axis_index("s") * 2 + lax.axis_index("c")
    base = wid * RPW
    pltpu.sync_copy(cand_hbm.at[pl.ds(base, RPW)], candb)

    def group(gid, carry):
        b = gid // 4
        sg = gid % 4
        u0 = b * N + base + sg * 16
        for t in range(16):
            nl = sg * 16 + t
            for j in range(KC // 16):
                m = candb[nl, pl.ds(j * 16, 16)]
                ridx[t, pl.ds(j * 16, 16)] = m + (b * N)
        copies = [pltpu.async_copy(kf_hbm.at[ridx.at[t]], kgb.at[t], sem)
                  for t in range(16)]
        for c in copies:
            c.wait()
        pltpu.sync_copy(kgb, kg_hbm.at[pl.ds(u0, 16)])
        return carry

    lax.fori_loop(0, B * 4, group, 0)


def _sc_route_body(w_hbm, cand_hbm, xf_hbm, imp_hbm, out_hbm,
                   candb, wbuf, ibuf, xvb, ridx, xgb, obuf, sem):
    wid = lax.axis_index("s") * 2 + lax.axis_index("c")
    base = wid * RPW
    pltpu.sync_copy(cand_hbm.at[pl.ds(base, RPW)], candb)
    lane = lax.iota(I32, 16)
    zi = lane * 0

    def group(gid, carry):
        b = gid // 4
        sg = gid % 4
        u0 = b * N + base + sg * 16
        pltpu.sync_copy(w_hbm.at[pl.ds(u0, 16)], wbuf)
        pltpu.sync_copy(imp_hbm.at[pl.ds(u0, 16)], ibuf)
        pltpu.sync_copy(xf_hbm.at[pl.ds(u0, 16)], xvb)
        for t in range(16):
            nl = sg * 16 + t
            for j in range(KC // 16):
                m = candb[nl, pl.ds(j * 16, 16)]
                ridx[t, pl.ds(j * 16, 16)] = m + (b * N)
        copies = [pltpu.async_copy(xf_hbm.at[ridx.at[t]], xgb.at[t], sem)
                  for t in range(16)]
        for c in copies:
            c.wait()
        iv = ibuf[pl.ds(0, 16)]

        def unit(t, c2):
            acc0 = jnp.zeros((16,), jnp.float32)
            acc1 = jnp.zeros((16,), jnp.float32)
            for j in range(KC // 16):
                wv = wbuf[t, pl.ds(j * 16, 16)]
                for k2 in range(16):
                    k = j * 16 + k2
                    wk = _gather16(wv, zi + k2)
                    acc0 = acc0 + wk * xgb[t, k, pl.ds(0, 16)]
                    acc1 = acc1 + wk * xgb[t, k, pl.ds(16, 16)]
            im = _gather16(iv, zi + t)
            om = np.float32(1.0) - im
            obuf[t, pl.ds(0, 16)] = im * acc0 + om * xvb[t, pl.ds(0, 16)]
            obuf[t, pl.ds(16, 16)] = im * acc1 + om * xvb[t, pl.ds(16, 16)]
            return c2

        lax.fori_loop(0, 16, unit, 0)
        pltpu.sync_copy(obuf, out_hbm.at[pl.ds(u0, 16)])
        return carry

    lax.fori_loop(0, B * 4, group, 0)


def _tc_dynw(kg_ref, q_ref, seg_ref, w_ref):
    kg = kg_ref[...]
    q = q_ref[...]
    R = q.shape[0]
    kg = kg.astype(jnp.bfloat16).astype(jnp.float32)
    q = q.astype(jnp.bfloat16).astype(jnp.float32)
    qt = jnp.concatenate([q] * KC, axis=1)
    v = jnp.dot(kg * qt, seg_ref[...],
                preferred_element_type=jnp.float32,
                precision=lax.Precision.HIGHEST) * np.float32(0.25)

    col = lax.broadcasted_iota(I32, (R, KC), 1)
    valid = col < TOPK_STA
    ib = lax.bitcast_convert_type(v, I32)
    keys = jnp.where(ib >= 0, ib, ib ^ jnp.int32(0x7FFFFFFF))
    keys = jnp.where(valid, keys, jnp.int32(NEG32))

    def body(i, c):
        lo, hi = c
        mid = _mid_floor(lo, hi) + ((lo ^ hi) & 1)
        cnt = jnp.sum((keys >= mid).astype(I32), axis=1, keepdims=True)
        ok = cnt >= TOPK_DYN
        return jnp.where(ok, mid, lo), jnp.where(ok, hi, mid - 1)

    lo = jnp.full((R, 1), NEG32, I32)
    hi = jnp.full((R, 1), POS32, I32)
    lo, hi = lax.fori_loop(0, 32, body, (lo, hi))
    t = lo
    c_gt = jnp.sum((keys > t).astype(I32), axis=1, keepdims=True)
    r = TOPK_DYN - c_gt
    tie = keys == t

    def body2(i, c):
        lo2, hi2 = c
        mid = _mid_floor(lo2, hi2)
        d = jnp.sum((tie & (col <= mid)).astype(I32), axis=1, keepdims=True)
        ok = d >= r
        return jnp.where(ok, lo2, mid + 1), jnp.where(ok, mid, hi2)

    lo2 = jnp.zeros((R, 1), I32)
    hi2 = jnp.full((R, 1), KC - 1, I32)
    lo2, hi2 = lax.fori_loop(0, 7, body2, (lo2, hi2))
    sel = (keys > t) | (tie & (col <= lo2))

    mx = jnp.max(jnp.where(sel, v, np.float32(-np.inf)), axis=1,
                 keepdims=True)
    e = jnp.exp(jnp.where(sel, v - mx, np.float32(-100.0)))
    e = jnp.where(sel, e, np.float32(0.0))
    w_ref[...] = e / jnp.sum(e, axis=1, keepdims=True)


def kernel(x, var_embed, W_t, b_t, W_q, b_q, W_k, b_k, W_g, b_g):
    f32 = jnp.float32
    x_var = jnp.transpose(x, (0, 2, 1))
    ve = var_embed[0]
    wt0 = W_t[:, 0][None]
    wt1 = W_t[:, 1][None]
    bt = b_t[None]
    wqt = W_q.T
    bq = b_q[None]
    wkt = W_k.T
    bk = b_k[None]
    bg = b_g.reshape(1, 1)

    sraw, qm, km, imp = pl.pallas_call(
        _tc_sim,
        out_shape=[
            jax.ShapeDtypeStruct((N, N), f32),
            jax.ShapeDtypeStruct((B, N, H), f32),
            jax.ShapeDtypeStruct((B, N, H), f32),
            jax.ShapeDtypeStruct((B, N, 1), f32),
        ],
    )(x_var, ve, wt0, wt1, bt, wqt, bq, wkt, bk, W_g, bg)

    s = pl.pallas_call(
        _tc_quant,
        out_shape=jax.ShapeDtypeStruct((N, N), f32),
    )(sraw)

    codes = pl.pallas_call(
        _tc_mask,
        grid=(N // _MB,),
        in_specs=[
            pl.BlockSpec((_MB, N), lambda i: (i, 0)),
            pl.BlockSpec((N, NC2), lambda i: (0, 0)),
        ],
        out_specs=pl.BlockSpec((_MB, NC2), lambda i: (i, 0)),
        out_shape=jax.ShapeDtypeStruct((N, NC2), I32),
    )(s, jnp.asarray(PACKM))

    mesh = plsc.VectorSubcoreMesh(core_axis_name="c", subcore_axis_name="s")

    cand = pl.kernel(
        _sc_compact_body,
        mesh=mesh,
        out_type=jax.ShapeDtypeStruct((N, KCP), I32),
        scratch_types=[
            pltpu.VMEM((16, NC2), I32),
            pltpu.VMEM((16, KCP), I32),
            pltpu.VMEM((256, 16), I32),
        ],
    )(codes, jnp.asarray(PTAB))

    kf = km.reshape(U, H)
    xvf = x_var.reshape(U, L)
    kg = pl.kernel(
        _sc_gather_body,
        mesh=mesh,
        compiler_params=pltpu.CompilerParams(use_tc_tiling_on_sc=False),
        out_type=jax.ShapeDtypeStruct((U, KC, H), f32),
        scratch_types=[
            pltpu.VMEM((RPW, KCP), I32),
            pltpu.VMEM((16, KC), I32),
            pltpu.VMEM((16, KC, H), f32),
            pltpu.SemaphoreType.DMA,
        ],
    )(cand, kf)

    seg = np.zeros((KC * H, KC), np.float32)
    for k in range(KC):
        seg[k * H:(k + 1) * H, k] = 1.0

    RB = 512
    w = pl.pallas_call(
        _tc_dynw,
        grid=(U // RB,),
        in_specs=[
            pl.BlockSpec((RB, KC * H), lambda i: (i, 0)),
            pl.BlockSpec((RB, H), lambda i: (i, 0)),
            pl.BlockSpec((KC * H, KC), lambda i: (0, 0)),
        ],
        out_specs=pl.BlockSpec((RB, KC), lambda i: (i, 0)),
        out_shape=jax.ShapeDtypeStruct((U, KC), f32),
    )(kg.reshape(U, KC * H), qm.reshape(U, H), jnp.asarray(seg))

    out_var = pl.kernel(
        _sc_route_body,
        mesh=mesh,
        compiler_params=pltpu.CompilerParams(use_tc_tiling_on_sc=False),
        out_type=jax.ShapeDtypeStruct((U, L), f32),
        scratch_types=[
            pltpu.VMEM((RPW, KCP), I32),
            pltpu.VMEM((16, KC), f32),
            pltpu.VMEM((16,), f32),
            pltpu.VMEM((16, L), f32),
            pltpu.VMEM((16, KC), I32),
            pltpu.VMEM((16, KC, L), f32),
            pltpu.VMEM((16, L), f32),
            pltpu.SemaphoreType.DMA,
        ],
    )(w, cand, xvf, imp.reshape(U))

    return jnp.transpose(out_var.reshape(B, N, L), (0, 2, 1))
